# baseline (speedup 1.0000x reference)
import jax
import jax.numpy as jnp
from jax.experimental import pallas as pl
from jax.experimental.pallas import tpu as pltpu

_BLK = 512
_K = 8


def _gate_stats_kernel(x_ref, wt_ref, b_ref, g_ref, stats_ref):
    i = pl.program_id(0)
    g = jnp.dot(x_ref[...], wt_ref[...], preferred_element_type=jnp.float32)
    g = g + b_ref[0:1, :]
    g_ref[...] = g
    blk_m = jnp.max(g, axis=0, keepdims=True)

    @pl.when(i == 0)
    def _():
        stats_ref[0:1, :] = blk_m
        stats_ref[1:2, :] = jnp.sum(jnp.exp(g - blk_m), axis=0, keepdims=True)

    @pl.when(i > 0)
    def _():
        m_old = stats_ref[0:1, :]
        m_new = jnp.maximum(m_old, blk_m)
        s_old = stats_ref[1:2, :]
        stats_ref[0:1, :] = m_new
        stats_ref[1:2, :] = s_old * jnp.exp(m_old - m_new) + jnp.sum(
            jnp.exp(g - m_new), axis=0, keepdims=True
        )


def _topk_kernel(g_ref, stats_ref, idx_ref):
    g = g_ref[...]
    m = stats_ref[0:1, :]
    s = stats_ref[1:2, :]
    p = jnp.exp(g - m) / s
    iota = jax.lax.broadcasted_iota(jnp.int32, p.shape, 1)
    n_exp = p.shape[1]
    vals = p
    cols = []
    for _ in range(_K):
        mx = jnp.max(vals, axis=1, keepdims=True)
        hit = vals == mx
        idx = jnp.min(jnp.where(hit, iota, n_exp), axis=1, keepdims=True)
        cols.append(idx)
        vals = jnp.where(iota == idx, -1.0, vals)
    idx_ref[...] = jnp.concatenate(cols, axis=1)


def kernel(x, W, b):
    n_tokens, d_model = x.shape
    n_exp = W.shape[0]
    wt = W.T
    b_pad = jnp.zeros((8, n_exp), jnp.float32).at[0, :].set(b)
    nb = n_tokens // _BLK

    g, stats = pl.pallas_call(
        _gate_stats_kernel,
        grid=(nb,),
        in_specs=[
            pl.BlockSpec((_BLK, d_model), lambda i: (i, 0)),
            pl.BlockSpec((d_model, n_exp), lambda i: (0, 0)),
            pl.BlockSpec((8, n_exp), lambda i: (0, 0)),
        ],
        out_specs=[
            pl.BlockSpec((_BLK, n_exp), lambda i: (i, 0)),
            pl.BlockSpec((8, n_exp), lambda i: (0, 0)),
        ],
        out_shape=[
            jax.ShapeDtypeStruct((n_tokens, n_exp), jnp.float32),
            jax.ShapeDtypeStruct((8, n_exp), jnp.float32),
        ],
    )(x, wt, b_pad)

    idx = pl.pallas_call(
        _topk_kernel,
        grid=(nb,),
        in_specs=[
            pl.BlockSpec((_BLK, n_exp), lambda i: (i, 0)),
            pl.BlockSpec((8, n_exp), lambda i: (0, 0)),
        ],
        out_specs=pl.BlockSpec((_BLK, _K), lambda i: (i, 0)),
        out_shape=jax.ShapeDtypeStruct((n_tokens, _K), jnp.int32),
    )(g, stats)
    return idx


# speedup vs baseline: 1.0075x; 1.0075x over previous
import jax
import jax.numpy as jnp
from jax.experimental import pallas as pl
from jax.experimental.pallas import tpu as pltpu

_BLK = 512
_K = 8


def _gate_stats_kernel(x_ref, wt_ref, b_ref, g_ref, stats_ref):
    g = jnp.dot(x_ref[...], wt_ref[...], preferred_element_type=jnp.float32)
    g = g + b_ref[0:1, :]
    g_ref[...] = g
    m = jnp.max(g, axis=0, keepdims=True)
    stats_ref[0:1, :] = m
    stats_ref[1:2, :] = jnp.sum(jnp.exp(g - m), axis=0, keepdims=True)


def _topk_kernel(g_ref, stats_ref, idx_ref):
    nb8, n_exp = stats_ref.shape
    stats = stats_ref[...].reshape(nb8 // 8, 8, n_exp)
    m_blk = stats[:, 0, :]
    s_blk = stats[:, 1, :]
    m = jnp.max(m_blk, axis=0, keepdims=True)
    s = jnp.sum(s_blk * jnp.exp(m_blk - m), axis=0, keepdims=True)
    c = m + jnp.log(s)

    score = g_ref[...] - c
    iota = jax.lax.broadcasted_iota(jnp.int32, score.shape, 1)
    cols = []
    for _ in range(_K):
        mx = jnp.max(score, axis=1, keepdims=True)
        hit = score == mx
        idx = jnp.min(jnp.where(hit, iota, n_exp), axis=1, keepdims=True)
        cols.append(idx)
        score = jnp.where(iota == idx, -jnp.inf, score)
    idx_ref[...] = jnp.concatenate(cols, axis=1)


def kernel(x, W, b):
    n_tokens, d_model = x.shape
    n_exp = W.shape[0]
    wt = W.T
    b_pad = jnp.zeros((8, n_exp), jnp.float32).at[0, :].set(b)
    nb = n_tokens // _BLK

    g, stats = pl.pallas_call(
        _gate_stats_kernel,
        grid=(nb,),
        in_specs=[
            pl.BlockSpec((_BLK, d_model), lambda i: (i, 0)),
            pl.BlockSpec((d_model, n_exp), lambda i: (0, 0)),
            pl.BlockSpec((8, n_exp), lambda i: (0, 0)),
        ],
        out_specs=[
            pl.BlockSpec((_BLK, n_exp), lambda i: (i, 0)),
            pl.BlockSpec((8, n_exp), lambda i: (i, 0)),
        ],
        out_shape=[
            jax.ShapeDtypeStruct((n_tokens, n_exp), jnp.float32),
            jax.ShapeDtypeStruct((nb * 8, n_exp), jnp.float32),
        ],
        compiler_params=pltpu.CompilerParams(
            dimension_semantics=("parallel",),
        ),
    )(x, wt, b_pad)

    idx = pl.pallas_call(
        _topk_kernel,
        grid=(nb,),
        in_specs=[
            pl.BlockSpec((_BLK, n_exp), lambda i: (i, 0)),
            pl.BlockSpec((nb * 8, n_exp), lambda i: (0, 0)),
        ],
        out_specs=pl.BlockSpec((_BLK, _K), lambda i: (i, 0)),
        out_shape=jax.ShapeDtypeStruct((n_tokens, _K), jnp.int32),
        compiler_params=pltpu.CompilerParams(
            dimension_semantics=("parallel",),
        ),
    )(g, stats)
    return idx
